# baseline (speedup 1.0000x reference)
import functools

import jax
import jax.numpy as jnp
from jax import lax
from jax.experimental import pallas as pl
from jax.experimental.pallas import tpu as pltpu
from jax.experimental.pallas import tpu_sc as plsc

N = 10000
E = 320000
D = 128
H = 8
C = 16
OUT = 64

NC = 2
NS = 16
NW = NC * NS
CB = 64
NCHUNK = E // CB
RPT = N // NS
TRIP = NCHUNK // NW
NSUP = TRIP // 3
NLEFT = NCHUNK - TRIP * NW


def _make_edge_kernel(ch, wcols):
  ngrp = ch // 16
  cw = ch + 16
  mesh = plsc.VectorSubcoreMesh(
      core_axis_name="c", subcore_axis_name="s", num_cores=NC, num_subcores=NS
  )

  @functools.partial(
      pl.kernel,
      out_type=jax.ShapeDtypeStruct((NC, N, cw), jnp.float32),
      mesh=mesh,
      compiler_params=pltpu.CompilerParams(
          use_tc_tiling_on_sc=False, needs_layout_passes=False),
      scratch_types=(
          [pltpu.VMEM((2, CB), jnp.int32)] * 3
          + [pltpu.VMEM((CB,), jnp.int32)] * 3
          + [pltpu.VMEM((CB, 16), jnp.float32)] * 3
          + [pltpu.VMEM((CB, cw), jnp.float32)] * 3
          + [pltpu.VMEM_SHARED((N, cw), jnp.float32)]
          + [pltpu.SemaphoreType.DMA] * 9
      ),
  )
  def edge_kernel(ei, hp, adst, zacc, oacc,
                  i0, i1, i2, x0, x1, x2, q0, q1, q2, m0, m1, m2,
                  acc_sh,
                  si0, si1, si2, sg0, sg1, sg2, ss0, ss1, ss2):
    idxs = [i0, i1, i2]
    sidx = [x0, x1, x2]
    a2s = [q0, q1, q2]
    ms = [m0, m1, m2]
    semi = [si0, si1, si2]
    semg = [sg0, sg1, sg2]
    sems = [ss0, ss1, ss2]

    cid = lax.axis_index("c")
    sid = lax.axis_index("s")
    wid = sid * NC + cid
    r0 = sid * RPT

    pltpu.sync_copy(zacc, acc_sh.at[pl.ds(r0, RPT)])
    plsc.subcore_barrier()

    colis = [jnp.full((16,), ch + wcols[j], jnp.int32) for j in range(ngrp)]

    def idx_start(j, s):
      pltpu.async_copy(ei.at[wid + j * NW], idxs[s], semi[s])

    def idx_wait(s):
      pltpu.make_async_copy(ei.at[0], idxs[s], semi[s]).wait()

    def gathers_start(s):
      pltpu.async_copy(hp.at[idxs[s].at[0]], ms[s], semg[s])
      pltpu.async_copy(adst.at[idxs[s].at[1]], a2s[s], semg[s])

    def gathers_wait(s):
      pltpu.make_async_copy(hp.at[idxs[s].at[0]], ms[s], semg[s]).wait()
      pltpu.make_async_copy(adst.at[idxs[s].at[1]], a2s[s], semg[s]).wait()

    def scatter_start(s):
      pltpu.async_copy(ms[s], acc_sh.at[sidx[s]], sems[s], add=True)

    def scatter_wait(s):
      pltpu.make_async_copy(ms[s], acc_sh.at[sidx[s]], sems[s]).wait()

    def copy_sidx(s):
      @plsc.parallel_loop(0, CB // 16, unroll=2)
      def cl(b):
        sidx[s][pl.ds(b * 16, 16)] = idxs[s][1, pl.ds(b * 16, 16)]

    def compute(s):
      a2_v, msg_v = a2s[s], ms[s]

      @plsc.parallel_loop(0, CB, unroll=4)
      def wloop(e):
        v = msg_v[e, pl.ds(ch, 16)] + a2_v[e, :]
        v = jnp.maximum(v, 0.2 * v)
        msg_v[e, pl.ds(ch, 16)] = jnp.exp(v)

      iota16 = lax.iota(jnp.int32, 16)

      @plsc.parallel_loop(0, CB // 16, unroll=2)
      def mloop(b):
        rows = jnp.full((16,), b * 16, jnp.int32) + iota16
        for j in range(ngrp):
          ws = plsc.load_gather(msg_v, [rows, colis[j]])
          for c in range(16):
            col = jnp.full((16,), j * 16 + c, jnp.int32)
            v = plsc.load_gather(msg_v, [rows, col])
            plsc.store_scatter(msg_v, [rows, col], ws * v)

    idx_start(0, 0)
    idx_wait(0)
    gathers_start(0)
    idx_start(1, 1)

    def superbody(t, carry):
      for k in range(3):
        s, s1, s2 = k, (k + 1) % 3, (k + 2) % 3
        j = t * 3 + k

        if k < 2:
          idx_wait(s1)
          pl.when(t > 0)(lambda: scatter_wait(s1))
          gathers_start(s1)
        else:
          scatter_wait(s1)

          def prefetch():
            idx_wait(s1)
            gathers_start(s1)

          pl.when(t < NSUP - 1)(prefetch)

        gathers_wait(s)

        if k == 0:
          idx_start(j + 2, s2)
        else:
          pl.when(t < NSUP - 1)(lambda: idx_start(j + 2, s2))

        copy_sidx(s)
        compute(s)
        scatter_start(s)
      return carry

    lax.fori_loop(0, NSUP, superbody, 0)
    scatter_wait(1)
    scatter_wait(2)

    @pl.when(wid < NLEFT)
    def leftover():
      idx_start(TRIP, 0)
      idx_wait(0)
      gathers_start(0)
      gathers_wait(0)
      copy_sidx(0)
      compute(0)
      scatter_start(0)
      scatter_wait(0)

    plsc.subcore_barrier()
    pltpu.sync_copy(acc_sh.at[pl.ds(r0, RPT)], oacc.at[cid, pl.ds(r0, RPT)])

  return edge_kernel


_edge128 = _make_edge_kernel(128, list(range(8)))
_edge64 = _make_edge_kernel(64, [0, 0, 0, 0])


def _tc1_body(x_ref, w_ref, am_ref, hp_ref, ad_ref):
  h = jnp.dot(x_ref[...], w_ref[...], preferred_element_type=jnp.float32)
  a = jnp.dot(h, am_ref[...], preferred_element_type=jnp.float32)
  hp_ref[...] = jnp.concatenate([h, a[:, :16]], axis=1)
  ad_ref[...] = a[:, 16:]


def _tc_mid_body(acc_ref, b_ref, g_ref, be_ref, r_ref, w_ref,
                 am_ref, res_ref, hact_ref, hp_ref, ad_ref, ch):
  asum = acc_ref[0] + acc_ref[1]
  nsum = asum[:, :ch]
  dsum = asum[:, ch:]
  rec = 1.0 / (dsum + 1e-16)
  rec_b = jnp.dot(rec, r_ref[...], preferred_element_type=jnp.float32)
  agg = nsum * rec_b + b_ref[...]
  a = g_ref[...] * agg + be_ref[...]
  hact = jnp.where(a >= 0, a, 0.01 * a)
  if res_ref is not None:
    hact = hact + res_ref[...]
  hlin = jnp.dot(hact, w_ref[...], preferred_element_type=jnp.float32)
  av = jnp.dot(hlin, am_ref[...], preferred_element_type=jnp.float32)
  if hact_ref is not None:
    hact_ref[...] = hact
  hp_ref[...] = jnp.concatenate([hlin, av[:, :16]], axis=1)
  ad_ref[...] = av[:, 16:]


def _tc2_body(acc_ref, b_ref, g_ref, be_ref, r_ref, w_ref, am_ref,
              hact_ref, hp_ref, ad_ref):
  _tc_mid_body(acc_ref, b_ref, g_ref, be_ref, r_ref, w_ref, am_ref,
               None, hact_ref, hp_ref, ad_ref, D)


def _tc3_body(acc_ref, b_ref, g_ref, be_ref, r_ref, w_ref, am_ref,
              res_ref, hp_ref, ad_ref):
  _tc_mid_body(acc_ref, b_ref, g_ref, be_ref, r_ref, w_ref, am_ref,
               res_ref, None, hp_ref, ad_ref, D)


def _tc4_body(acc_ref, b_ref, out_ref):
  asum = acc_ref[0] + acc_ref[1]
  nsum = asum[:, :OUT]
  dsum = asum[:, OUT:]
  rec = 1.0 / (dsum + 1e-16)
  out_ref[...] = nsum * rec[:, 0:1] + b_ref[...]


_f32 = jnp.float32

_tc1 = pl.pallas_call(
    _tc1_body,
    out_shape=(
        jax.ShapeDtypeStruct((N, D + 16), _f32),
        jax.ShapeDtypeStruct((N, 16), _f32),
    ),
)

_tc2 = pl.pallas_call(
    _tc2_body,
    out_shape=(
        jax.ShapeDtypeStruct((N, D), _f32),
        jax.ShapeDtypeStruct((N, D + 16), _f32),
        jax.ShapeDtypeStruct((N, 16), _f32),
    ),
)

_tc3 = pl.pallas_call(
    _tc3_body,
    out_shape=(
        jax.ShapeDtypeStruct((N, OUT + 16), _f32),
        jax.ShapeDtypeStruct((N, 16), _f32),
    ),
)

_tc4 = pl.pallas_call(
    _tc4_body,
    out_shape=jax.ShapeDtypeStruct((N, OUT), _f32),
)


def kernel(x, edge_index, W1, as1, ad1, b1, g1, be1, W2, as2, ad2, b2, g2,
           be2, W3, as3, ad3, b3):
  ei = edge_index.astype(jnp.int32).reshape(2, NCHUNK, CB).transpose(1, 0, 2)

  mask = (jnp.arange(128)[:, None] // 16 == (jnp.arange(16)[None, :] % 8))
  mask = mask.astype(_f32)
  am1 = jnp.concatenate(
      [as1.reshape(128, 1) * mask, ad1.reshape(128, 1) * mask], axis=1)
  am2 = jnp.concatenate(
      [as2.reshape(128, 1) * mask, ad2.reshape(128, 1) * mask], axis=1)
  am3 = jnp.concatenate(
      [jnp.tile(as3.reshape(64, 1), (1, 16)),
       jnp.tile(ad3.reshape(64, 1), (1, 16))], axis=1)

  r16 = (jnp.arange(128)[None, :] // 16 == jnp.arange(16)[:, None])
  r16 = r16.astype(_f32) * (jnp.arange(16)[:, None] < 8).astype(_f32)

  z144 = jnp.zeros((RPT, D + 16), _f32)
  z80 = jnp.zeros((RPT, OUT + 16), _f32)

  b1r = b1.reshape(1, D)
  g1r = g1.reshape(1, D)
  be1r = be1.reshape(1, D)
  b2r = b2.reshape(1, D)
  g2r = g2.reshape(1, D)
  be2r = be2.reshape(1, D)
  b3r = b3.reshape(1, OUT)

  hp1, d1 = _tc1(x, W1, am1)
  acc1 = _edge128(ei, hp1, d1, z144)
  h1a, hp2, d2 = _tc2(acc1, b1r, g1r, be1r, r16, W2, am2)
  acc2 = _edge128(ei, hp2, d2, z144)
  hp3, d3 = _tc3(acc2, b2r, g2r, be2r, r16, W3, am3, h1a)
  acc3 = _edge64(ei, hp3, d3, z80)
  return _tc4(acc3, b3r)


# speedup vs baseline: 2.7002x; 2.7002x over previous
import functools

import jax
import jax.numpy as jnp
from jax import lax
from jax.experimental import pallas as pl
from jax.experimental.pallas import tpu as pltpu
from jax.experimental.pallas import tpu_sc as plsc

N = 10000
E = 320000
D = 128
H = 8
C = 16
OUT = 64

NC = 2
NS = 16
NW = NC * NS
CB = 64
NCHUNK = E // CB
RPT = N // NS
TRIP = NCHUNK // NW
NSUP = TRIP // 3
NLEFT = NCHUNK - TRIP * NW


def _make_edge_kernel(ch, wcols):
  ngrp = ch // 16
  cw = ch + 16
  mesh = plsc.VectorSubcoreMesh(
      core_axis_name="c", subcore_axis_name="s", num_cores=NC, num_subcores=NS
  )

  @functools.partial(
      pl.kernel,
      out_type=jax.ShapeDtypeStruct((NC, N, cw), jnp.float32),
      mesh=mesh,
      compiler_params=pltpu.CompilerParams(
          use_tc_tiling_on_sc=False, needs_layout_passes=False),
      scratch_types=(
          [pltpu.VMEM((2, CB), jnp.int32)] * 3
          + [pltpu.VMEM((CB,), jnp.int32)] * 3
          + [pltpu.VMEM((CB, 16), jnp.float32)] * 3
          + [pltpu.VMEM((CB, cw), jnp.float32)] * 3
          + [pltpu.VMEM_SHARED((N, cw), jnp.float32)]
          + [pltpu.SemaphoreType.DMA] * 9
      ),
  )
  def edge_kernel(ei, hp, adst, zacc, oacc,
                  i0, i1, i2, x0, x1, x2, q0, q1, q2, m0, m1, m2,
                  acc_sh,
                  si0, si1, si2, sg0, sg1, sg2, ss0, ss1, ss2):
    idxs = [i0, i1, i2]
    sidx = [x0, x1, x2]
    a2s = [q0, q1, q2]
    ms = [m0, m1, m2]
    semi = [si0, si1, si2]
    semg = [sg0, sg1, sg2]
    sems = [ss0, ss1, ss2]

    cid = lax.axis_index("c")
    sid = lax.axis_index("s")
    wid = sid * NC + cid
    r0 = sid * RPT

    pltpu.sync_copy(zacc, acc_sh.at[pl.ds(r0, RPT)])
    plsc.subcore_barrier()

    colis = [jnp.full((16,), ch + wcols[j], jnp.int32) for j in range(ngrp)]

    def idx_start(j, s):
      pltpu.async_copy(ei.at[wid + j * NW], idxs[s], semi[s])

    def idx_wait(s):
      pltpu.make_async_copy(ei.at[0], idxs[s], semi[s]).wait()

    def gathers_start(s):
      pltpu.async_copy(hp.at[idxs[s].at[0]], ms[s], semg[s])
      pltpu.async_copy(adst.at[idxs[s].at[1]], a2s[s], semg[s])

    def gathers_wait(s):
      pltpu.make_async_copy(hp.at[idxs[s].at[0]], ms[s], semg[s]).wait()
      pltpu.make_async_copy(adst.at[idxs[s].at[1]], a2s[s], semg[s]).wait()

    def scatter_start(s):
      pltpu.async_copy(ms[s], acc_sh.at[sidx[s]], sems[s], add=True)

    def scatter_wait(s):
      pltpu.make_async_copy(ms[s], acc_sh.at[sidx[s]], sems[s]).wait()

    def copy_sidx(s):
      @plsc.parallel_loop(0, CB // 16, unroll=2)
      def cl(b):
        sidx[s][pl.ds(b * 16, 16)] = idxs[s][1, pl.ds(b * 16, 16)]

    def compute(s):
      a2_v, msg_v = a2s[s], ms[s]

      @plsc.parallel_loop(0, CB, unroll=4)
      def wloop(e):
        v = msg_v[e, pl.ds(ch, 16)] + a2_v[e, :]
        v = jnp.maximum(v, 0.2 * v)
        msg_v[e, pl.ds(ch, 16)] = jnp.exp(v)

      @plsc.parallel_loop(0, CB, unroll=8)
      def mloop(e):
        rowi = jnp.full((16,), e, jnp.int32)
        for j in range(ngrp):
          ws = plsc.load_gather(msg_v, [rowi, colis[j]])
          msg_v[e, pl.ds(j * 16, 16)] = ws * msg_v[e, pl.ds(j * 16, 16)]

    idx_start(0, 0)
    idx_wait(0)
    gathers_start(0)
    idx_start(1, 1)

    def superbody(t, carry):
      for k in range(3):
        s, s1, s2 = k, (k + 1) % 3, (k + 2) % 3
        j = t * 3 + k

        if k < 2:
          idx_wait(s1)
          pl.when(t > 0)(lambda: scatter_wait(s1))
          gathers_start(s1)
        else:
          scatter_wait(s1)

          def prefetch():
            idx_wait(s1)
            gathers_start(s1)

          pl.when(t < NSUP - 1)(prefetch)

        gathers_wait(s)

        if k == 0:
          idx_start(j + 2, s2)
        else:
          pl.when(t < NSUP - 1)(lambda: idx_start(j + 2, s2))

        copy_sidx(s)
        compute(s)
        scatter_start(s)
      return carry

    lax.fori_loop(0, NSUP, superbody, 0)
    scatter_wait(1)
    scatter_wait(2)

    @pl.when(wid < NLEFT)
    def leftover():
      idx_start(TRIP, 0)
      idx_wait(0)
      gathers_start(0)
      gathers_wait(0)
      copy_sidx(0)
      compute(0)
      scatter_start(0)
      scatter_wait(0)

    plsc.subcore_barrier()
    pltpu.sync_copy(acc_sh.at[pl.ds(r0, RPT)], oacc.at[cid, pl.ds(r0, RPT)])

  return edge_kernel


_edge128 = _make_edge_kernel(128, list(range(8)))
_edge64 = _make_edge_kernel(64, [0, 0, 0, 0])


def _tc1_body(x_ref, w_ref, am_ref, hp_ref, ad_ref):
  h = jnp.dot(x_ref[...], w_ref[...], preferred_element_type=jnp.float32)
  a = jnp.dot(h, am_ref[...], preferred_element_type=jnp.float32)
  hp_ref[...] = jnp.concatenate([h, a[:, :16]], axis=1)
  ad_ref[...] = a[:, 16:]


def _tc_mid_body(acc_ref, b_ref, g_ref, be_ref, r_ref, w_ref,
                 am_ref, res_ref, hact_ref, hp_ref, ad_ref, ch):
  asum = acc_ref[0] + acc_ref[1]
  nsum = asum[:, :ch]
  dsum = asum[:, ch:]
  rec = 1.0 / (dsum + 1e-16)
  rec_b = jnp.dot(rec, r_ref[...], preferred_element_type=jnp.float32)
  agg = nsum * rec_b + b_ref[...]
  a = g_ref[...] * agg + be_ref[...]
  hact = jnp.where(a >= 0, a, 0.01 * a)
  if res_ref is not None:
    hact = hact + res_ref[...]
  hlin = jnp.dot(hact, w_ref[...], preferred_element_type=jnp.float32)
  av = jnp.dot(hlin, am_ref[...], preferred_element_type=jnp.float32)
  if hact_ref is not None:
    hact_ref[...] = hact
  hp_ref[...] = jnp.concatenate([hlin, av[:, :16]], axis=1)
  ad_ref[...] = av[:, 16:]


def _tc2_body(acc_ref, b_ref, g_ref, be_ref, r_ref, w_ref, am_ref,
              hact_ref, hp_ref, ad_ref):
  _tc_mid_body(acc_ref, b_ref, g_ref, be_ref, r_ref, w_ref, am_ref,
               None, hact_ref, hp_ref, ad_ref, D)


def _tc3_body(acc_ref, b_ref, g_ref, be_ref, r_ref, w_ref, am_ref,
              res_ref, hp_ref, ad_ref):
  _tc_mid_body(acc_ref, b_ref, g_ref, be_ref, r_ref, w_ref, am_ref,
               res_ref, None, hp_ref, ad_ref, D)


def _tc4_body(acc_ref, b_ref, out_ref):
  asum = acc_ref[0] + acc_ref[1]
  nsum = asum[:, :OUT]
  dsum = asum[:, OUT:]
  rec = 1.0 / (dsum + 1e-16)
  out_ref[...] = nsum * rec[:, 0:1] + b_ref[...]


_f32 = jnp.float32

_tc1 = pl.pallas_call(
    _tc1_body,
    out_shape=(
        jax.ShapeDtypeStruct((N, D + 16), _f32),
        jax.ShapeDtypeStruct((N, 16), _f32),
    ),
)

_tc2 = pl.pallas_call(
    _tc2_body,
    out_shape=(
        jax.ShapeDtypeStruct((N, D), _f32),
        jax.ShapeDtypeStruct((N, D + 16), _f32),
        jax.ShapeDtypeStruct((N, 16), _f32),
    ),
)

_tc3 = pl.pallas_call(
    _tc3_body,
    out_shape=(
        jax.ShapeDtypeStruct((N, OUT + 16), _f32),
        jax.ShapeDtypeStruct((N, 16), _f32),
    ),
)

_tc4 = pl.pallas_call(
    _tc4_body,
    out_shape=jax.ShapeDtypeStruct((N, OUT), _f32),
)


def kernel(x, edge_index, W1, as1, ad1, b1, g1, be1, W2, as2, ad2, b2, g2,
           be2, W3, as3, ad3, b3):
  ei = edge_index.astype(jnp.int32).reshape(2, NCHUNK, CB).transpose(1, 0, 2)

  mask = (jnp.arange(128)[:, None] // 16 == (jnp.arange(16)[None, :] % 8))
  mask = mask.astype(_f32)
  am1 = jnp.concatenate(
      [as1.reshape(128, 1) * mask, ad1.reshape(128, 1) * mask], axis=1)
  am2 = jnp.concatenate(
      [as2.reshape(128, 1) * mask, ad2.reshape(128, 1) * mask], axis=1)
  am3 = jnp.concatenate(
      [jnp.tile(as3.reshape(64, 1), (1, 16)),
       jnp.tile(ad3.reshape(64, 1), (1, 16))], axis=1)

  r16 = (jnp.arange(128)[None, :] // 16 == jnp.arange(16)[:, None])
  r16 = r16.astype(_f32) * (jnp.arange(16)[:, None] < 8).astype(_f32)

  z144 = jnp.zeros((RPT, D + 16), _f32)
  z80 = jnp.zeros((RPT, OUT + 16), _f32)

  b1r = b1.reshape(1, D)
  g1r = g1.reshape(1, D)
  be1r = be1.reshape(1, D)
  b2r = b2.reshape(1, D)
  g2r = g2.reshape(1, D)
  be2r = be2.reshape(1, D)
  b3r = b3.reshape(1, OUT)

  hp1, d1 = _tc1(x, W1, am1)
  acc1 = _edge128(ei, hp1, d1, z144)
  h1a, hp2, d2 = _tc2(acc1, b1r, g1r, be1r, r16, W2, am2)
  acc2 = _edge128(ei, hp2, d2, z144)
  hp3, d3 = _tc3(acc2, b2r, g2r, be2r, r16, W3, am3, h1a)
  acc3 = _edge64(ei, hp3, d3, z80)
  return _tc4(acc3, b3r)


# speedup vs baseline: 2.7861x; 1.0318x over previous
import functools

import jax
import jax.numpy as jnp
from jax import lax
from jax.experimental import pallas as pl
from jax.experimental.pallas import tpu as pltpu
from jax.experimental.pallas import tpu_sc as plsc

N = 10000
E = 320000
D = 128
H = 8
C = 16
OUT = 64

NC = 2
NS = 16
NW = NC * NS
CB = 64
NCHUNK = E // CB
RPT = N // NS
TRIP = NCHUNK // NW
NSUP = TRIP // 3
NLEFT = NCHUNK - TRIP * NW


def _make_edge_kernel(ch, wcols):
  ngrp = ch // 16
  cw = ch + 16
  mesh = plsc.VectorSubcoreMesh(
      core_axis_name="c", subcore_axis_name="s", num_cores=NC, num_subcores=NS
  )

  @functools.partial(
      pl.kernel,
      out_type=jax.ShapeDtypeStruct((NC, N, cw), jnp.float32),
      mesh=mesh,
      compiler_params=pltpu.CompilerParams(
          use_tc_tiling_on_sc=False, needs_layout_passes=False),
      scratch_types=(
          [pltpu.VMEM((2, CB), jnp.int32)] * 3
          + [pltpu.VMEM((CB,), jnp.int32)] * 3
          + [pltpu.VMEM((CB, 16), jnp.float32)] * 3
          + [pltpu.VMEM((CB, cw), jnp.float32)] * 3
          + [pltpu.VMEM_SHARED((N, cw), jnp.float32)]
          + [pltpu.SemaphoreType.DMA] * 9
      ),
  )
  def edge_kernel(ei, hp, adst, zacc, oacc,
                  i0, i1, i2, x0, x1, x2, q0, q1, q2, m0, m1, m2,
                  acc_sh,
                  si0, si1, si2, sg0, sg1, sg2, ss0, ss1, ss2):
    idxs = [i0, i1, i2]
    sidx = [x0, x1, x2]
    a2s = [q0, q1, q2]
    ms = [m0, m1, m2]
    semi = [si0, si1, si2]
    semg = [sg0, sg1, sg2]
    sems = [ss0, ss1, ss2]

    cid = lax.axis_index("c")
    sid = lax.axis_index("s")
    wid = sid * NC + cid
    r0 = sid * RPT

    pltpu.sync_copy(zacc, acc_sh.at[pl.ds(r0, RPT)])
    plsc.subcore_barrier()

    colis = [jnp.full((16,), wcols[j], jnp.int32) for j in range(ngrp)]

    def lane_splat(x, idx):
      return lax.gather(
          x, idx[:, None],
          lax.GatherDimensionNumbers(
              offset_dims=(), collapsed_slice_dims=(0,), start_index_map=(0,)),
          (1,), mode=lax.GatherScatterMode.PROMISE_IN_BOUNDS)

    def idx_start(j, s):
      pltpu.async_copy(ei.at[wid + j * NW], idxs[s], semi[s])

    def idx_wait(s):
      pltpu.make_async_copy(ei.at[0], idxs[s], semi[s]).wait()

    def gathers_start(s):
      pltpu.async_copy(hp.at[idxs[s].at[0]], ms[s], semg[s])
      pltpu.async_copy(adst.at[idxs[s].at[1]], a2s[s], semg[s])

    def gathers_wait(s):
      pltpu.make_async_copy(hp.at[idxs[s].at[0]], ms[s], semg[s]).wait()
      pltpu.make_async_copy(adst.at[idxs[s].at[1]], a2s[s], semg[s]).wait()

    def scatter_start(s):
      pltpu.async_copy(ms[s], acc_sh.at[sidx[s]], sems[s], add=True)

    def scatter_wait(s):
      pltpu.make_async_copy(ms[s], acc_sh.at[sidx[s]], sems[s]).wait()

    def copy_sidx(s):
      @plsc.parallel_loop(0, CB // 16, unroll=2)
      def cl(b):
        sidx[s][pl.ds(b * 16, 16)] = idxs[s][1, pl.ds(b * 16, 16)]

    def compute(s):
      a2_v, msg_v = a2s[s], ms[s]

      @plsc.parallel_loop(0, CB, unroll=4)
      def wloop(e):
        v = msg_v[e, pl.ds(ch, 16)] + a2_v[e, :]
        v = jnp.maximum(v, 0.2 * v)
        msg_v[e, pl.ds(ch, 16)] = jnp.exp(v)

      @plsc.parallel_loop(0, CB, unroll=4)
      def mloop(e):
        wrow = msg_v[e, pl.ds(ch, 16)]
        for j in range(ngrp):
          ws = lane_splat(wrow, colis[j])
          msg_v[e, pl.ds(j * 16, 16)] = ws * msg_v[e, pl.ds(j * 16, 16)]

    idx_start(0, 0)
    idx_wait(0)
    gathers_start(0)
    idx_start(1, 1)

    def superbody(t, carry):
      for k in range(3):
        s, s1, s2 = k, (k + 1) % 3, (k + 2) % 3
        j = t * 3 + k

        if k < 2:
          idx_wait(s1)
          pl.when(t > 0)(lambda: scatter_wait(s1))
          gathers_start(s1)
        else:
          scatter_wait(s1)

          def prefetch():
            idx_wait(s1)
            gathers_start(s1)

          pl.when(t < NSUP - 1)(prefetch)

        gathers_wait(s)

        if k == 0:
          idx_start(j + 2, s2)
        else:
          pl.when(t < NSUP - 1)(lambda: idx_start(j + 2, s2))

        copy_sidx(s)
        compute(s)
        scatter_start(s)
      return carry

    lax.fori_loop(0, NSUP, superbody, 0)
    scatter_wait(1)
    scatter_wait(2)

    @pl.when(wid < NLEFT)
    def leftover():
      idx_start(TRIP, 0)
      idx_wait(0)
      gathers_start(0)
      gathers_wait(0)
      copy_sidx(0)
      compute(0)
      scatter_start(0)
      scatter_wait(0)

    plsc.subcore_barrier()
    pltpu.sync_copy(acc_sh.at[pl.ds(r0, RPT)], oacc.at[cid, pl.ds(r0, RPT)])

  return edge_kernel


_edge128 = _make_edge_kernel(128, list(range(8)))
_edge64 = _make_edge_kernel(64, [0, 0, 0, 0])


def _tc1_body(x_ref, w_ref, am_ref, hp_ref, ad_ref):
  h = jnp.dot(x_ref[...], w_ref[...], preferred_element_type=jnp.float32)
  a = jnp.dot(h, am_ref[...], preferred_element_type=jnp.float32)
  hp_ref[...] = jnp.concatenate([h, a[:, :16]], axis=1)
  ad_ref[...] = a[:, 16:]


def _tc_mid_body(acc_ref, b_ref, g_ref, be_ref, r_ref, w_ref,
                 am_ref, res_ref, hact_ref, hp_ref, ad_ref, ch):
  asum = acc_ref[0] + acc_ref[1]
  nsum = asum[:, :ch]
  dsum = asum[:, ch:]
  rec = 1.0 / (dsum + 1e-16)
  rec_b = jnp.dot(rec, r_ref[...], preferred_element_type=jnp.float32)
  agg = nsum * rec_b + b_ref[...]
  a = g_ref[...] * agg + be_ref[...]
  hact = jnp.where(a >= 0, a, 0.01 * a)
  if res_ref is not None:
    hact = hact + res_ref[...]
  hlin = jnp.dot(hact, w_ref[...], preferred_element_type=jnp.float32)
  av = jnp.dot(hlin, am_ref[...], preferred_element_type=jnp.float32)
  if hact_ref is not None:
    hact_ref[...] = hact
  hp_ref[...] = jnp.concatenate([hlin, av[:, :16]], axis=1)
  ad_ref[...] = av[:, 16:]


def _tc2_body(acc_ref, b_ref, g_ref, be_ref, r_ref, w_ref, am_ref,
              hact_ref, hp_ref, ad_ref):
  _tc_mid_body(acc_ref, b_ref, g_ref, be_ref, r_ref, w_ref, am_ref,
               None, hact_ref, hp_ref, ad_ref, D)


def _tc3_body(acc_ref, b_ref, g_ref, be_ref, r_ref, w_ref, am_ref,
              res_ref, hp_ref, ad_ref):
  _tc_mid_body(acc_ref, b_ref, g_ref, be_ref, r_ref, w_ref, am_ref,
               res_ref, None, hp_ref, ad_ref, D)


def _tc4_body(acc_ref, b_ref, out_ref):
  asum = acc_ref[0] + acc_ref[1]
  nsum = asum[:, :OUT]
  dsum = asum[:, OUT:]
  rec = 1.0 / (dsum + 1e-16)
  out_ref[...] = nsum * rec[:, 0:1] + b_ref[...]


_f32 = jnp.float32

_tc1 = pl.pallas_call(
    _tc1_body,
    out_shape=(
        jax.ShapeDtypeStruct((N, D + 16), _f32),
        jax.ShapeDtypeStruct((N, 16), _f32),
    ),
)

_tc2 = pl.pallas_call(
    _tc2_body,
    out_shape=(
        jax.ShapeDtypeStruct((N, D), _f32),
        jax.ShapeDtypeStruct((N, D + 16), _f32),
        jax.ShapeDtypeStruct((N, 16), _f32),
    ),
)

_tc3 = pl.pallas_call(
    _tc3_body,
    out_shape=(
        jax.ShapeDtypeStruct((N, OUT + 16), _f32),
        jax.ShapeDtypeStruct((N, 16), _f32),
    ),
)

_tc4 = pl.pallas_call(
    _tc4_body,
    out_shape=jax.ShapeDtypeStruct((N, OUT), _f32),
)


def kernel(x, edge_index, W1, as1, ad1, b1, g1, be1, W2, as2, ad2, b2, g2,
           be2, W3, as3, ad3, b3):
  ei = edge_index.astype(jnp.int32).reshape(2, NCHUNK, CB).transpose(1, 0, 2)

  mask = (jnp.arange(128)[:, None] // 16 == (jnp.arange(16)[None, :] % 8))
  mask = mask.astype(_f32)
  am1 = jnp.concatenate(
      [as1.reshape(128, 1) * mask, ad1.reshape(128, 1) * mask], axis=1)
  am2 = jnp.concatenate(
      [as2.reshape(128, 1) * mask, ad2.reshape(128, 1) * mask], axis=1)
  am3 = jnp.concatenate(
      [jnp.tile(as3.reshape(64, 1), (1, 16)),
       jnp.tile(ad3.reshape(64, 1), (1, 16))], axis=1)

  r16 = (jnp.arange(128)[None, :] // 16 == jnp.arange(16)[:, None])
  r16 = r16.astype(_f32) * (jnp.arange(16)[:, None] < 8).astype(_f32)

  z144 = jnp.zeros((RPT, D + 16), _f32)
  z80 = jnp.zeros((RPT, OUT + 16), _f32)

  b1r = b1.reshape(1, D)
  g1r = g1.reshape(1, D)
  be1r = be1.reshape(1, D)
  b2r = b2.reshape(1, D)
  g2r = g2.reshape(1, D)
  be2r = be2.reshape(1, D)
  b3r = b3.reshape(1, OUT)

  hp1, d1 = _tc1(x, W1, am1)
  acc1 = _edge128(ei, hp1, d1, z144)
  h1a, hp2, d2 = _tc2(acc1, b1r, g1r, be1r, r16, W2, am2)
  acc2 = _edge128(ei, hp2, d2, z144)
  hp3, d3 = _tc3(acc2, b2r, g2r, be2r, r16, W3, am3, h1a)
  acc3 = _edge64(ei, hp3, d3, z80)
  return _tc4(acc3, b3r)
